# baseline (speedup 1.0000x reference)
import functools

import jax
import jax.numpy as jnp
from jax import lax
from jax.experimental import pallas as pl
from jax.experimental.pallas import tpu as pltpu
from jax.experimental.pallas import tpu_sc as plsc

_K = 65536
_DIM = 128
_B = 16384
_PTR = 60000
_TAIL = _K - _PTR
_HEAD = _B - _TAIL
_MID = _PTR - _HEAD

_SC_INFO = plsc.get_sparse_core_info()
_NC = _SC_INFO.num_cores


def _z_body(z_new, z_buf, z_out, sem):
    copies = (
        pltpu.make_async_copy(
            z_new.at[pl.ds(0, _TAIL)], z_out.at[pl.ds(_PTR, _TAIL)], sem.at[0]),
        pltpu.make_async_copy(
            z_new.at[pl.ds(_TAIL, _HEAD)], z_out.at[pl.ds(0, _HEAD)], sem.at[1]),
        pltpu.make_async_copy(
            z_buf.at[pl.ds(_HEAD, _MID)], z_out.at[pl.ds(_HEAD, _MID)], sem.at[2]),
    )
    for c in copies:
        c.start()
    for c in copies:
        c.wait()


def _sc_body(t_new, e_new, b_new, t_buf, e_buf, b_buf,
             t_out, e_out, b_out, fscr, iscr):
    wid = lax.axis_index("s") * _NC + lax.axis_index("c")
    tasks = []
    for new, buf, out, scr in ((t_new, t_buf, t_out, fscr),
                               (e_new, e_buf, e_out, fscr),
                               (b_new, b_buf, b_out, iscr)):
        tasks.append((new, 0, out, _PTR, _TAIL, scr))
        tasks.append((new, _TAIL, out, 0, _HEAD, scr))
        tasks.append((buf, _HEAD, out, _HEAD, _MID, scr))
    for k, (src, so, dst, do, n, scr) in enumerate(tasks):
        @pl.when(wid == k)
        def _():
            pltpu.sync_copy(src.at[pl.ds(so, n)], scr.at[pl.ds(0, n)])
            pltpu.sync_copy(scr.at[pl.ds(0, n)], dst.at[pl.ds(do, n)])


_sc_enqueue = functools.partial(
    pl.kernel,
    out_type=(
        jax.ShapeDtypeStruct((_K,), jnp.float32),
        jax.ShapeDtypeStruct((_K,), jnp.float32),
        jax.ShapeDtypeStruct((_K,), jnp.int32),
    ),
    mesh=plsc.VectorSubcoreMesh(core_axis_name="c", subcore_axis_name="s"),
    scratch_types=[
        pltpu.VMEM((_MID,), jnp.float32),
        pltpu.VMEM((_MID,), jnp.int32),
    ],
)(_sc_body)


def kernel(z_new, t_new, e_new, b_new, z_buf, t_buf, e_buf, b_buf):
    z = pl.pallas_call(
        _z_body,
        out_shape=jax.ShapeDtypeStruct((_K, _DIM), jnp.float32),
        in_specs=[pl.BlockSpec(memory_space=pltpu.HBM)] * 2,
        out_specs=pl.BlockSpec(memory_space=pltpu.HBM),
        scratch_shapes=[pltpu.SemaphoreType.DMA((3,))],
    )(z_new, z_buf)
    t, e, b = _sc_enqueue(t_new, e_new, b_new, t_buf, e_buf, b_buf)
    new_ptr = jnp.asarray((_PTR + _B) % _K, dtype=jnp.int32)
    new_size = jnp.asarray(min(_B, _K), dtype=jnp.int32)
    return (z, t, e, b, new_ptr, new_size)


# speedup vs baseline: 1.0559x; 1.0559x over previous
import functools

import jax
import jax.numpy as jnp
from jax import lax
from jax.experimental import pallas as pl
from jax.experimental.pallas import tpu as pltpu
from jax.experimental.pallas import tpu_sc as plsc

_K = 65536
_DIM = 128
_B = 16384
_PTR = 60000
_TAIL = _K - _PTR
_HEAD = _B - _TAIL
_MID = _PTR - _HEAD

_SC_INFO = plsc.get_sparse_core_info()
_NC = _SC_INFO.num_cores


_ZR = 32
_NB = _K // _ZR
_HEAD_B = _HEAD // _ZR
_PTR_B = _PTR // _ZR
_TAIL_SHIFT = _PTR_B
_HEAD_SHIFT = _TAIL // _ZR


def _z_new_idx(i):
    return (jnp.where(i < _HEAD_B, i + _HEAD_SHIFT,
                      jnp.where(i >= _PTR_B, i - _TAIL_SHIFT, 0)), 0)


def _z_buf_idx(i):
    in_mid = (i >= _HEAD_B) & (i < _PTR_B)
    return (jnp.where(in_mid, i, 0), 0)


def _z_body(new_ref, buf_ref, out_ref):
    i = pl.program_id(0)
    from_new = (i < _HEAD_B) | (i >= _PTR_B)
    out_ref[...] = jnp.where(from_new, new_ref[...], buf_ref[...])


def _sc_body(t_new, e_new, b_new, t_buf, e_buf, b_buf,
             t_out, e_out, b_out, fscr, iscr):
    wid = lax.axis_index("s") * _NC + lax.axis_index("c")
    tasks = []
    for new, buf, out, scr in ((t_new, t_buf, t_out, fscr),
                               (e_new, e_buf, e_out, fscr),
                               (b_new, b_buf, b_out, iscr)):
        tasks.append((new, 0, out, _PTR, _TAIL, scr))
        tasks.append((new, _TAIL, out, 0, _HEAD, scr))
        tasks.append((buf, _HEAD, out, _HEAD, _MID, scr))
    for k, (src, so, dst, do, n, scr) in enumerate(tasks):
        @pl.when(wid == k)
        def _():
            pltpu.sync_copy(src.at[pl.ds(so, n)], scr.at[pl.ds(0, n)])
            pltpu.sync_copy(scr.at[pl.ds(0, n)], dst.at[pl.ds(do, n)])


_sc_enqueue = functools.partial(
    pl.kernel,
    out_type=(
        jax.ShapeDtypeStruct((_K,), jnp.float32),
        jax.ShapeDtypeStruct((_K,), jnp.float32),
        jax.ShapeDtypeStruct((_K,), jnp.int32),
    ),
    mesh=plsc.VectorSubcoreMesh(core_axis_name="c", subcore_axis_name="s"),
    scratch_types=[
        pltpu.VMEM((_MID,), jnp.float32),
        pltpu.VMEM((_MID,), jnp.int32),
    ],
)(_sc_body)


def kernel(z_new, t_new, e_new, b_new, z_buf, t_buf, e_buf, b_buf):
    z = pl.pallas_call(
        _z_body,
        grid=(_NB,),
        out_shape=jax.ShapeDtypeStruct((_K, _DIM), jnp.float32),
        in_specs=[
            pl.BlockSpec((_ZR, _DIM), _z_new_idx),
            pl.BlockSpec((_ZR, _DIM), _z_buf_idx),
        ],
        out_specs=pl.BlockSpec((_ZR, _DIM), lambda i: (i, 0)),
    )(z_new, z_buf)
    t, e, b = _sc_enqueue(t_new, e_new, b_new, t_buf, e_buf, b_buf)
    new_ptr = jnp.asarray((_PTR + _B) % _K, dtype=jnp.int32)
    new_size = jnp.asarray(min(_B, _K), dtype=jnp.int32)
    return (z, t, e, b, new_ptr, new_size)


# speedup vs baseline: 14.2955x; 13.5382x over previous
import functools

import jax
import jax.numpy as jnp
from jax import lax
from jax.experimental import pallas as pl
from jax.experimental.pallas import tpu as pltpu
from jax.experimental.pallas import tpu_sc as plsc

_K = 65536
_DIM = 128
_B = 16384
_PTR = 60000
_TAIL = _K - _PTR
_HEAD = _B - _TAIL
_MID = _PTR - _HEAD

_SC_INFO = plsc.get_sparse_core_info()
_NC = _SC_INFO.num_cores


_CH = 1024
_DEPTH = 4

_Z_CHUNKS = []
for _off in range(0, _TAIL, _CH):
    _Z_CHUNKS.append((0, _off, _PTR + _off, min(_CH, _TAIL - _off)))
for _off in range(0, _HEAD, _CH):
    _Z_CHUNKS.append((0, _TAIL + _off, _off, min(_CH, _HEAD - _off)))
for _off in range(0, _MID, _CH):
    _Z_CHUNKS.append((1, _HEAD + _off, _HEAD + _off, min(_CH, _MID - _off)))


def _z_body(z_new, z_buf, z_out, scratch, in_sem, out_sem):
    srcs = (z_new, z_buf)
    n_chunks = len(_Z_CHUNKS)

    def in_copy(k):
        s, so, _, n = _Z_CHUNKS[k]
        return pltpu.make_async_copy(
            srcs[s].at[pl.ds(so, n)],
            scratch.at[k % _DEPTH, pl.ds(0, n)],
            in_sem.at[k % _DEPTH])

    def out_copy(k):
        _, _, do, n = _Z_CHUNKS[k]
        return pltpu.make_async_copy(
            scratch.at[k % _DEPTH, pl.ds(0, n)],
            z_out.at[pl.ds(do, n)],
            out_sem.at[k % _DEPTH])

    for k in range(min(_DEPTH, n_chunks)):
        in_copy(k).start()
    for k in range(n_chunks):
        in_copy(k).wait()
        out_copy(k).start()
        if k + _DEPTH < n_chunks:
            out_copy(k).wait()
            in_copy(k + _DEPTH).start()
    for k in range(max(0, n_chunks - _DEPTH), n_chunks):
        out_copy(k).wait()


def _sc_body(t_new, e_new, b_new, t_buf, e_buf, b_buf,
             t_out, e_out, b_out, fscr, iscr):
    wid = lax.axis_index("s") * _NC + lax.axis_index("c")
    tasks = []
    for new, buf, out, scr in ((t_new, t_buf, t_out, fscr),
                               (e_new, e_buf, e_out, fscr),
                               (b_new, b_buf, b_out, iscr)):
        tasks.append((new, 0, out, _PTR, _TAIL, scr))
        tasks.append((new, _TAIL, out, 0, _HEAD, scr))
        tasks.append((buf, _HEAD, out, _HEAD, _MID, scr))
    for k, (src, so, dst, do, n, scr) in enumerate(tasks):
        @pl.when(wid == k)
        def _():
            pltpu.sync_copy(src.at[pl.ds(so, n)], scr.at[pl.ds(0, n)])
            pltpu.sync_copy(scr.at[pl.ds(0, n)], dst.at[pl.ds(do, n)])


_sc_enqueue = functools.partial(
    pl.kernel,
    out_type=(
        jax.ShapeDtypeStruct((_K,), jnp.float32),
        jax.ShapeDtypeStruct((_K,), jnp.float32),
        jax.ShapeDtypeStruct((_K,), jnp.int32),
    ),
    mesh=plsc.VectorSubcoreMesh(core_axis_name="c", subcore_axis_name="s"),
    scratch_types=[
        pltpu.VMEM((_MID,), jnp.float32),
        pltpu.VMEM((_MID,), jnp.int32),
    ],
)(_sc_body)


def kernel(z_new, t_new, e_new, b_new, z_buf, t_buf, e_buf, b_buf):
    z = pl.pallas_call(
        _z_body,
        out_shape=jax.ShapeDtypeStruct((_K, _DIM), jnp.float32),
        in_specs=[pl.BlockSpec(memory_space=pltpu.HBM)] * 2,
        out_specs=pl.BlockSpec(memory_space=pltpu.HBM),
        scratch_shapes=[
            pltpu.VMEM((_DEPTH, _CH, _DIM), jnp.float32),
            pltpu.SemaphoreType.DMA((_DEPTH,)),
            pltpu.SemaphoreType.DMA((_DEPTH,)),
        ],
    )(z_new, z_buf)
    t, e, b = _sc_enqueue(t_new, e_new, b_new, t_buf, e_buf, b_buf)
    new_ptr = jnp.asarray((_PTR + _B) % _K, dtype=jnp.int32)
    new_size = jnp.asarray(min(_B, _K), dtype=jnp.int32)
    return (z, t, e, b, new_ptr, new_size)


# speedup vs baseline: 19.4180x; 1.3583x over previous
import functools

import jax
import jax.numpy as jnp
from jax import lax
from jax.experimental import pallas as pl
from jax.experimental.pallas import tpu as pltpu
from jax.experimental.pallas import tpu_sc as plsc

_K = 65536
_DIM = 128
_B = 16384
_PTR = 60000
_TAIL = _K - _PTR
_HEAD = _B - _TAIL
_MID = _PTR - _HEAD

_SC_INFO = plsc.get_sparse_core_info()
_NC = _SC_INFO.num_cores


_CH = 2048
_DEPTH = 4

_Z_CHUNKS = []
for _off in range(0, _TAIL, _CH):
    _Z_CHUNKS.append((0, _off, _PTR + _off, min(_CH, _TAIL - _off)))
for _off in range(0, _HEAD, _CH):
    _Z_CHUNKS.append((0, _TAIL + _off, _off, min(_CH, _HEAD - _off)))
for _off in range(0, _MID, _CH):
    _Z_CHUNKS.append((1, _HEAD + _off, _HEAD + _off, min(_CH, _MID - _off)))


def _z_body(z_new, z_buf, z_out, scratch, in_sem, out_sem):
    srcs = (z_new, z_buf)
    n_chunks = len(_Z_CHUNKS)

    def in_copy(k):
        s, so, _, n = _Z_CHUNKS[k]
        return pltpu.make_async_copy(
            srcs[s].at[pl.ds(so, n)],
            scratch.at[k % _DEPTH, pl.ds(0, n)],
            in_sem.at[k % _DEPTH])

    def out_copy(k):
        _, _, do, n = _Z_CHUNKS[k]
        return pltpu.make_async_copy(
            scratch.at[k % _DEPTH, pl.ds(0, n)],
            z_out.at[pl.ds(do, n)],
            out_sem.at[k % _DEPTH])

    for k in range(min(_DEPTH, n_chunks)):
        in_copy(k).start()
    for k in range(n_chunks):
        in_copy(k).wait()
        out_copy(k).start()
        if k + _DEPTH < n_chunks:
            out_copy(k).wait()
            in_copy(k + _DEPTH).start()
    for k in range(max(0, n_chunks - _DEPTH), n_chunks):
        out_copy(k).wait()


def _sc_body(t_new, e_new, b_new, t_buf, e_buf, b_buf,
             t_out, e_out, b_out, fscr, iscr):
    wid = lax.axis_index("s") * _NC + lax.axis_index("c")
    tasks = []
    for new, buf, out, scr in ((t_new, t_buf, t_out, fscr),
                               (e_new, e_buf, e_out, fscr),
                               (b_new, b_buf, b_out, iscr)):
        tasks.append((new, 0, out, _PTR, _TAIL, scr))
        tasks.append((new, _TAIL, out, 0, _HEAD, scr))
        tasks.append((buf, _HEAD, out, _HEAD, _MID, scr))
    for k, (src, so, dst, do, n, scr) in enumerate(tasks):
        @pl.when(wid == k)
        def _():
            pltpu.sync_copy(src.at[pl.ds(so, n)], scr.at[pl.ds(0, n)])
            pltpu.sync_copy(scr.at[pl.ds(0, n)], dst.at[pl.ds(do, n)])


_sc_enqueue = functools.partial(
    pl.kernel,
    out_type=(
        jax.ShapeDtypeStruct((_K,), jnp.float32),
        jax.ShapeDtypeStruct((_K,), jnp.float32),
        jax.ShapeDtypeStruct((_K,), jnp.int32),
    ),
    mesh=plsc.VectorSubcoreMesh(core_axis_name="c", subcore_axis_name="s"),
    scratch_types=[
        pltpu.VMEM((_MID,), jnp.float32),
        pltpu.VMEM((_MID,), jnp.int32),
    ],
)(_sc_body)


def kernel(z_new, t_new, e_new, b_new, z_buf, t_buf, e_buf, b_buf):
    z = pl.pallas_call(
        _z_body,
        out_shape=jax.ShapeDtypeStruct((_K, _DIM), jnp.float32),
        in_specs=[pl.BlockSpec(memory_space=pltpu.HBM)] * 2,
        out_specs=pl.BlockSpec(memory_space=pltpu.HBM),
        scratch_shapes=[
            pltpu.VMEM((_DEPTH, _CH, _DIM), jnp.float32),
            pltpu.SemaphoreType.DMA((_DEPTH,)),
            pltpu.SemaphoreType.DMA((_DEPTH,)),
        ],
    )(z_new, z_buf)
    t, e, b = _sc_enqueue(t_new, e_new, b_new, t_buf, e_buf, b_buf)
    new_ptr = jnp.asarray((_PTR + _B) % _K, dtype=jnp.int32)
    new_size = jnp.asarray(min(_B, _K), dtype=jnp.int32)
    return (z, t, e, b, new_ptr, new_size)


# speedup vs baseline: 23.7524x; 1.2232x over previous
import functools

import jax
import jax.numpy as jnp
from jax import lax
from jax.experimental import pallas as pl
from jax.experimental.pallas import tpu as pltpu
from jax.experimental.pallas import tpu_sc as plsc

_K = 65536
_DIM = 128
_B = 16384
_PTR = 60000
_TAIL = _K - _PTR
_HEAD = _B - _TAIL
_MID = _PTR - _HEAD

_SC_INFO = plsc.get_sparse_core_info()
_NC = _SC_INFO.num_cores


_CH = 4096
_DEPTH = 4

_Z_CHUNKS = []
for _off in range(0, _TAIL, _CH):
    _Z_CHUNKS.append((0, _off, _PTR + _off, min(_CH, _TAIL - _off)))
for _off in range(0, _HEAD, _CH):
    _Z_CHUNKS.append((0, _TAIL + _off, _off, min(_CH, _HEAD - _off)))
for _off in range(0, _MID, _CH):
    _Z_CHUNKS.append((1, _HEAD + _off, _HEAD + _off, min(_CH, _MID - _off)))


def _z_body(z_new, z_buf, z_out, scratch, in_sem, out_sem):
    srcs = (z_new, z_buf)
    n_chunks = len(_Z_CHUNKS)

    def in_copy(k):
        s, so, _, n = _Z_CHUNKS[k]
        return pltpu.make_async_copy(
            srcs[s].at[pl.ds(so, n)],
            scratch.at[k % _DEPTH, pl.ds(0, n)],
            in_sem.at[k % _DEPTH])

    def out_copy(k):
        _, _, do, n = _Z_CHUNKS[k]
        return pltpu.make_async_copy(
            scratch.at[k % _DEPTH, pl.ds(0, n)],
            z_out.at[pl.ds(do, n)],
            out_sem.at[k % _DEPTH])

    for k in range(min(_DEPTH, n_chunks)):
        in_copy(k).start()
    for k in range(n_chunks):
        in_copy(k).wait()
        out_copy(k).start()
        if k + _DEPTH < n_chunks:
            out_copy(k).wait()
            in_copy(k + _DEPTH).start()
    for k in range(max(0, n_chunks - _DEPTH), n_chunks):
        out_copy(k).wait()


def _sc_body(t_new, e_new, b_new, t_buf, e_buf, b_buf,
             t_out, e_out, b_out, fscr, iscr):
    wid = lax.axis_index("s") * _NC + lax.axis_index("c")
    tasks = []
    for new, buf, out, scr in ((t_new, t_buf, t_out, fscr),
                               (e_new, e_buf, e_out, fscr),
                               (b_new, b_buf, b_out, iscr)):
        tasks.append((new, 0, out, _PTR, _TAIL, scr))
        tasks.append((new, _TAIL, out, 0, _HEAD, scr))
        tasks.append((buf, _HEAD, out, _HEAD, _MID, scr))
    for k, (src, so, dst, do, n, scr) in enumerate(tasks):
        @pl.when(wid == k)
        def _():
            pltpu.sync_copy(src.at[pl.ds(so, n)], scr.at[pl.ds(0, n)])
            pltpu.sync_copy(scr.at[pl.ds(0, n)], dst.at[pl.ds(do, n)])


_sc_enqueue = functools.partial(
    pl.kernel,
    out_type=(
        jax.ShapeDtypeStruct((_K,), jnp.float32),
        jax.ShapeDtypeStruct((_K,), jnp.float32),
        jax.ShapeDtypeStruct((_K,), jnp.int32),
    ),
    mesh=plsc.VectorSubcoreMesh(core_axis_name="c", subcore_axis_name="s"),
    scratch_types=[
        pltpu.VMEM((_MID,), jnp.float32),
        pltpu.VMEM((_MID,), jnp.int32),
    ],
)(_sc_body)


def kernel(z_new, t_new, e_new, b_new, z_buf, t_buf, e_buf, b_buf):
    z = pl.pallas_call(
        _z_body,
        out_shape=jax.ShapeDtypeStruct((_K, _DIM), jnp.float32),
        in_specs=[pl.BlockSpec(memory_space=pltpu.HBM)] * 2,
        out_specs=pl.BlockSpec(memory_space=pltpu.HBM),
        scratch_shapes=[
            pltpu.VMEM((_DEPTH, _CH, _DIM), jnp.float32),
            pltpu.SemaphoreType.DMA((_DEPTH,)),
            pltpu.SemaphoreType.DMA((_DEPTH,)),
        ],
    )(z_new, z_buf)
    t, e, b = _sc_enqueue(t_new, e_new, b_new, t_buf, e_buf, b_buf)
    new_ptr = jnp.asarray((_PTR + _B) % _K, dtype=jnp.int32)
    new_size = jnp.asarray(min(_B, _K), dtype=jnp.int32)
    return (z, t, e, b, new_ptr, new_size)


# speedup vs baseline: 25.9898x; 1.0942x over previous
import functools

import jax
import jax.numpy as jnp
from jax import lax
from jax.experimental import pallas as pl
from jax.experimental.pallas import tpu as pltpu
from jax.experimental.pallas import tpu_sc as plsc

_K = 65536
_DIM = 128
_B = 16384
_PTR = 60000
_TAIL = _K - _PTR
_HEAD = _B - _TAIL
_MID = _PTR - _HEAD

_SC_INFO = plsc.get_sparse_core_info()
_NC = _SC_INFO.num_cores


_CH = 8192
_DEPTH = 4

_Z_CHUNKS = []
for _off in range(0, _TAIL, _CH):
    _Z_CHUNKS.append((0, _off, _PTR + _off, min(_CH, _TAIL - _off)))
for _off in range(0, _HEAD, _CH):
    _Z_CHUNKS.append((0, _TAIL + _off, _off, min(_CH, _HEAD - _off)))
for _off in range(0, _MID, _CH):
    _Z_CHUNKS.append((1, _HEAD + _off, _HEAD + _off, min(_CH, _MID - _off)))


def _z_body(z_new, z_buf, z_out, scratch, in_sem, out_sem):
    srcs = (z_new, z_buf)
    n_chunks = len(_Z_CHUNKS)

    def in_copy(k):
        s, so, _, n = _Z_CHUNKS[k]
        return pltpu.make_async_copy(
            srcs[s].at[pl.ds(so, n)],
            scratch.at[k % _DEPTH, pl.ds(0, n)],
            in_sem.at[k % _DEPTH])

    def out_copy(k):
        _, _, do, n = _Z_CHUNKS[k]
        return pltpu.make_async_copy(
            scratch.at[k % _DEPTH, pl.ds(0, n)],
            z_out.at[pl.ds(do, n)],
            out_sem.at[k % _DEPTH])

    for k in range(min(_DEPTH, n_chunks)):
        in_copy(k).start()
    for k in range(n_chunks):
        in_copy(k).wait()
        out_copy(k).start()
        if k + _DEPTH < n_chunks:
            out_copy(k).wait()
            in_copy(k + _DEPTH).start()
    for k in range(max(0, n_chunks - _DEPTH), n_chunks):
        out_copy(k).wait()


def _sc_body(t_new, e_new, b_new, t_buf, e_buf, b_buf,
             t_out, e_out, b_out, fscr, iscr):
    wid = lax.axis_index("s") * _NC + lax.axis_index("c")
    tasks = []
    for new, buf, out, scr in ((t_new, t_buf, t_out, fscr),
                               (e_new, e_buf, e_out, fscr),
                               (b_new, b_buf, b_out, iscr)):
        tasks.append((new, 0, out, _PTR, _TAIL, scr))
        tasks.append((new, _TAIL, out, 0, _HEAD, scr))
        tasks.append((buf, _HEAD, out, _HEAD, _MID, scr))
    for k, (src, so, dst, do, n, scr) in enumerate(tasks):
        @pl.when(wid == k)
        def _():
            pltpu.sync_copy(src.at[pl.ds(so, n)], scr.at[pl.ds(0, n)])
            pltpu.sync_copy(scr.at[pl.ds(0, n)], dst.at[pl.ds(do, n)])


_sc_enqueue = functools.partial(
    pl.kernel,
    out_type=(
        jax.ShapeDtypeStruct((_K,), jnp.float32),
        jax.ShapeDtypeStruct((_K,), jnp.float32),
        jax.ShapeDtypeStruct((_K,), jnp.int32),
    ),
    mesh=plsc.VectorSubcoreMesh(core_axis_name="c", subcore_axis_name="s"),
    scratch_types=[
        pltpu.VMEM((_MID,), jnp.float32),
        pltpu.VMEM((_MID,), jnp.int32),
    ],
)(_sc_body)


def kernel(z_new, t_new, e_new, b_new, z_buf, t_buf, e_buf, b_buf):
    z = pl.pallas_call(
        _z_body,
        out_shape=jax.ShapeDtypeStruct((_K, _DIM), jnp.float32),
        in_specs=[pl.BlockSpec(memory_space=pltpu.HBM)] * 2,
        out_specs=pl.BlockSpec(memory_space=pltpu.HBM),
        scratch_shapes=[
            pltpu.VMEM((_DEPTH, _CH, _DIM), jnp.float32),
            pltpu.SemaphoreType.DMA((_DEPTH,)),
            pltpu.SemaphoreType.DMA((_DEPTH,)),
        ],
    )(z_new, z_buf)
    t, e, b = _sc_enqueue(t_new, e_new, b_new, t_buf, e_buf, b_buf)
    new_ptr = jnp.asarray((_PTR + _B) % _K, dtype=jnp.int32)
    new_size = jnp.asarray(min(_B, _K), dtype=jnp.int32)
    return (z, t, e, b, new_ptr, new_size)
